# baseline (speedup 1.0000x reference)
import jax
import jax.numpy as jnp
from jax import lax
from jax.experimental import pallas as pl
from jax.experimental.pallas import tpu as pltpu

_LANES = 128
_SUB = 8


def _nms_kernel(conf_ref, anc_ref, pr_ref,
                sc_out, bx_out, cid_out,
                s_ref, y1_ref, x1_ref, y2_ref, x2_ref, area_ref, idx_ref,
                *, max_boxes, score_thr, iou_thr, rows, pair):
    c = pl.program_id(0)
    neg_inf = jnp.float32(-jnp.inf)
    big = jnp.int32(rows * _LANES)

    y1_ref[...] = anc_ref[0] + pr_ref[0]
    x1_ref[...] = anc_ref[1] + pr_ref[1]
    y2_ref[...] = anc_ref[2] + pr_ref[2]
    x2_ref[...] = anc_ref[3] + pr_ref[3]
    area_ref[...] = (y2_ref[...] - y1_ref[...]) * (x2_ref[...] - x1_ref[...])
    idx_ref[...] = (lax.broadcasted_iota(jnp.int32, (rows, _LANES), 0) * _LANES
                    + lax.broadcasted_iota(jnp.int32, (rows, _LANES), 1))

    sc_out[...] = jnp.zeros_like(sc_out)
    bx_out[...] = jnp.zeros_like(bx_out)
    cid_out[...] = jnp.zeros_like(cid_out)

    lane1 = lax.broadcasted_iota(jnp.int32, (1, 1, _LANES), 2)
    sub8 = lax.broadcasted_iota(jnp.int32, (_SUB, _LANES), 0)
    lane8 = lax.broadcasted_iota(jnp.int32, (_SUB, _LANES), 1)
    j_iota = lax.broadcasted_iota(jnp.int32, (1, 4, 1), 1)

    def first_argmax(p):
        s0 = jnp.where(conf_ref[p] >= score_thr, conf_ref[p], neg_inf)
        s_ref[p] = s0
        colmax = jnp.max(s0, axis=0, keepdims=True)
        rowhit = jnp.min(jnp.where(s0 == colmax, idx_ref[...], big),
                         axis=0, keepdims=True)
        v = jnp.max(colmax)
        idx = jnp.min(jnp.where(colmax == v, rowhit, big))
        return v, idx

    init = tuple(first_argmax(p) for p in range(pair))
    init = tuple(x for vi in init for x in vi)

    def one_class(p, t, v, idx):
        r = idx // _LANES
        l = idx % _LANES
        rbase = pl.multiple_of((r >> 3) << 3, _SUB)
        pick_m = (sub8 == (r & 7)) & (lane8 == l)

        def pick(ref):
            tile = ref[pl.ds(rbase, _SUB), :]
            return jnp.sum(jnp.where(pick_m, tile, 0.0))

        by1 = pick(y1_ref)
        bx1 = pick(x1_ref)
        by2 = pick(y2_ref)
        bx2 = pick(x2_ref)
        keep = v > neg_inf

        s = s_ref[p]
        iy1 = jnp.maximum(by1, y1_ref[...])
        ix1 = jnp.maximum(bx1, x1_ref[...])
        iy2 = jnp.minimum(by2, y2_ref[...])
        ix2 = jnp.minimum(bx2, x2_ref[...])
        inter = jnp.maximum(iy2 - iy1, 0.0) * jnp.maximum(ix2 - ix1, 0.0)
        area_a = (by2 - by1) * (bx2 - bx1)
        union = area_a + area_ref[...] - inter
        iou = jnp.where(union > 0.0, inter / union, 0.0)
        news = jnp.where((iou > iou_thr) | (idx_ref[...] == idx), neg_inf, s)
        s_ref[p] = news

        colmax = jnp.max(news, axis=0, keepdims=True)
        rowhit = jnp.min(jnp.where(news == colmax, idx_ref[...], big),
                         axis=0, keepdims=True)
        v2 = jnp.max(colmax)
        idx2 = jnp.min(jnp.where(colmax == v2, rowhit, big))

        sel = lane1 == t
        sc_out[p] = jnp.where(sel[0], jnp.where(keep, v, 0.0), sc_out[p])
        cid_out[p] = jnp.where(sel[0], jnp.where(keep, c * pair + p + 1, 0),
                               cid_out[p])
        coords = jnp.where(j_iota == 0, by1,
                           jnp.where(j_iota == 1, bx1,
                                     jnp.where(j_iota == 2, by2, bx2)))
        coords = jnp.where(keep, coords, 0.0)
        bx_out[pl.ds(p, 1)] = jnp.where(sel, coords, bx_out[pl.ds(p, 1)])
        return v2, idx2

    def body(t, carry):
        out = []
        for p in range(pair):
            v, idx = carry[2 * p], carry[2 * p + 1]
            out.extend(one_class(p, t, v, idx))
        return tuple(out)

    lax.fori_loop(0, max_boxes, body, init)


def _run_nms(confidence, anchors_all, pr, max_boxes, score_thr, iou_thr):
    n, num_classes = confidence.shape
    rows = ((n + _LANES * _SUB - 1) // (_LANES * _SUB)) * _SUB
    n_pad = rows * _LANES
    pair = 1

    conf_t = jnp.pad(confidence.T, ((0, 0), (0, n_pad - n)))
    conf_t = conf_t.reshape(num_classes, rows, _LANES)
    anc_t = jnp.pad(anchors_all.T, ((0, 0), (0, n_pad - n))).reshape(4, rows, _LANES)
    pr_t = jnp.pad(pr.T, ((0, 0), (0, n_pad - n))).reshape(4, rows, _LANES)

    out_lanes = ((max_boxes + _LANES - 1) // _LANES) * _LANES

    def kern(*refs):
        return _nms_kernel(*refs, max_boxes=max_boxes, score_thr=score_thr,
                           iou_thr=iou_thr, rows=rows, pair=pair)

    sc, bx, cid = pl.pallas_call(
        kern,
        grid=(num_classes // pair,),
        in_specs=[
            pl.BlockSpec((pair, rows, _LANES), lambda c: (c, 0, 0)),
            pl.BlockSpec((4, rows, _LANES), lambda c: (0, 0, 0)),
            pl.BlockSpec((4, rows, _LANES), lambda c: (0, 0, 0)),
        ],
        out_specs=[
            pl.BlockSpec((pair, 1, out_lanes), lambda c: (c, 0, 0)),
            pl.BlockSpec((pair, 4, out_lanes), lambda c: (c, 0, 0)),
            pl.BlockSpec((pair, 1, out_lanes), lambda c: (c, 0, 0)),
        ],
        out_shape=[
            jax.ShapeDtypeStruct((num_classes, 1, out_lanes), jnp.float32),
            jax.ShapeDtypeStruct((num_classes, 4, out_lanes), jnp.float32),
            jax.ShapeDtypeStruct((num_classes, 1, out_lanes), jnp.int32),
        ],
        scratch_shapes=[
            pltpu.VMEM((pair, rows, _LANES), jnp.float32),
            pltpu.VMEM((rows, _LANES), jnp.float32),
            pltpu.VMEM((rows, _LANES), jnp.float32),
            pltpu.VMEM((rows, _LANES), jnp.float32),
            pltpu.VMEM((rows, _LANES), jnp.float32),
            pltpu.VMEM((rows, _LANES), jnp.float32),
            pltpu.VMEM((rows, _LANES), jnp.int32),
        ],
        compiler_params=pltpu.CompilerParams(
            dimension_semantics=("parallel",),
        ),
    )(conf_t, anc_t, pr_t)

    sel_scores = sc[:, 0, :max_boxes]
    sel_boxes = jnp.transpose(bx[:, :, :max_boxes], (0, 2, 1))
    class_id = cid[:, 0, :max_boxes]
    return sel_scores, sel_boxes, class_id


def kernel(confidence, anchors_all, pr):
    return _run_nms(confidence, anchors_all, pr,
                    max_boxes=100, score_thr=0.5, iou_thr=0.45)


# speedup vs baseline: 1.2299x; 1.2299x over previous
import jax
import jax.numpy as jnp
from jax import lax
from jax.experimental import pallas as pl
from jax.experimental.pallas import tpu as pltpu

_LANES = 128
_SUB = 8


def _nms_kernel(conf_ref, anc_ref, pr_ref,
                sc_out, bx_out, cid_out,
                s_ref, y1_ref, x1_ref, y2_ref, x2_ref, area_ref, idx_ref,
                *, max_boxes, score_thr, iou_thr, rows, pair):
    c = pl.program_id(0)
    neg_inf = jnp.float32(-jnp.inf)
    big = jnp.int32(rows * _LANES)

    y1_ref[...] = anc_ref[0] + pr_ref[0]
    x1_ref[...] = anc_ref[1] + pr_ref[1]
    y2_ref[...] = anc_ref[2] + pr_ref[2]
    x2_ref[...] = anc_ref[3] + pr_ref[3]
    area_ref[...] = (y2_ref[...] - y1_ref[...]) * (x2_ref[...] - x1_ref[...])
    idx_ref[...] = (lax.broadcasted_iota(jnp.int32, (rows, _LANES), 0) * _LANES
                    + lax.broadcasted_iota(jnp.int32, (rows, _LANES), 1))

    sc_out[...] = jnp.zeros_like(sc_out)
    bx_out[...] = jnp.zeros_like(bx_out)
    cid_out[...] = jnp.zeros_like(cid_out)

    lane1 = lax.broadcasted_iota(jnp.int32, (1, 1, _LANES), 2)
    sub8 = lax.broadcasted_iota(jnp.int32, (_SUB, _LANES), 0)
    lane8 = lax.broadcasted_iota(jnp.int32, (_SUB, _LANES), 1)
    j_iota = lax.broadcasted_iota(jnp.int32, (1, 4, 1), 1)
    piota = lax.broadcasted_iota(jnp.int32, (pair, 1, 1), 0)

    def vec_p(vals):
        acc = vals[-1]
        for p in reversed(range(len(vals) - 1)):
            acc = jnp.where(piota == p, vals[p], acc)
        return acc

    def reduce_argmax(news, idx3):
        colmax = jnp.max(news, axis=1, keepdims=True)
        rowhit = jnp.min(jnp.where(news == colmax, idx3, big),
                         axis=1, keepdims=True)
        v = jnp.max(colmax, axis=2, keepdims=True)
        idxv = jnp.min(jnp.where(colmax == v, rowhit, big),
                       axis=2, keepdims=True)
        return v, idxv

    idx3 = idx_ref[...][None]

    s0 = jnp.where(conf_ref[...] >= score_thr, conf_ref[...], neg_inf)
    s_ref[...] = s0
    init = reduce_argmax(s0, idx3)

    def body(t, carry):
        v, idxv = carry

        by1s, bx1s, by2s, bx2s = [], [], [], []
        for p in range(pair):
            idx_p = idxv[p, 0, 0]
            r = idx_p // _LANES
            l = idx_p % _LANES
            rbase = pl.multiple_of((r >> 3) << 3, _SUB)
            pick_m = (sub8 == (r & 7)) & (lane8 == l)

            def pick(ref):
                tile = ref[pl.ds(rbase, _SUB), :]
                return jnp.sum(jnp.where(pick_m, tile, 0.0))

            by1s.append(pick(y1_ref))
            bx1s.append(pick(x1_ref))
            by2s.append(pick(y2_ref))
            bx2s.append(pick(x2_ref))

        by1 = vec_p(by1s)
        bx1 = vec_p(bx1s)
        by2 = vec_p(by2s)
        bx2 = vec_p(bx2s)
        keep = v > neg_inf

        s = s_ref[...]
        iy1 = jnp.maximum(by1, y1_ref[...][None])
        ix1 = jnp.maximum(bx1, x1_ref[...][None])
        iy2 = jnp.minimum(by2, y2_ref[...][None])
        ix2 = jnp.minimum(bx2, x2_ref[...][None])
        inter = jnp.maximum(iy2 - iy1, 0.0) * jnp.maximum(ix2 - ix1, 0.0)
        area_a = (by2 - by1) * (bx2 - bx1)
        union = area_a + area_ref[...][None] - inter
        iou = jnp.where(union > 0.0, inter / union, 0.0)
        news = jnp.where((iou > iou_thr) | (idx3 == idxv), neg_inf, s)
        s_ref[...] = news

        nxt = reduce_argmax(news, idx3)

        sel = lane1 == t
        sc_out[...] = jnp.where(sel, jnp.where(keep, v, 0.0), sc_out[...])
        cidv = jnp.where(keep, c * pair + piota + 1, 0)
        cid_out[...] = jnp.where(sel, cidv, cid_out[...])
        coords = jnp.where(j_iota == 0, by1,
                           jnp.where(j_iota == 1, bx1,
                                     jnp.where(j_iota == 2, by2, bx2)))
        coords = jnp.where(keep, coords, 0.0)
        bx_out[...] = jnp.where(sel, coords, bx_out[...])
        return nxt

    lax.fori_loop(0, max_boxes, body, init)


def _run_nms(confidence, anchors_all, pr, max_boxes, score_thr, iou_thr,
             pair=2):
    n, num_classes = confidence.shape
    rows = ((n + _LANES * _SUB - 1) // (_LANES * _SUB)) * _SUB
    n_pad = rows * _LANES
    if num_classes % pair != 0:
        pair = 1

    conf_t = jnp.pad(confidence.T, ((0, 0), (0, n_pad - n)))
    conf_t = conf_t.reshape(num_classes, rows, _LANES)
    anc_t = jnp.pad(anchors_all.T, ((0, 0), (0, n_pad - n))).reshape(4, rows, _LANES)
    pr_t = jnp.pad(pr.T, ((0, 0), (0, n_pad - n))).reshape(4, rows, _LANES)

    out_lanes = ((max_boxes + _LANES - 1) // _LANES) * _LANES

    def kern(*refs):
        return _nms_kernel(*refs, max_boxes=max_boxes, score_thr=score_thr,
                           iou_thr=iou_thr, rows=rows, pair=pair)

    sc, bx, cid = pl.pallas_call(
        kern,
        grid=(num_classes // pair,),
        in_specs=[
            pl.BlockSpec((pair, rows, _LANES), lambda c: (c, 0, 0)),
            pl.BlockSpec((4, rows, _LANES), lambda c: (0, 0, 0)),
            pl.BlockSpec((4, rows, _LANES), lambda c: (0, 0, 0)),
        ],
        out_specs=[
            pl.BlockSpec((pair, 1, out_lanes), lambda c: (c, 0, 0)),
            pl.BlockSpec((pair, 4, out_lanes), lambda c: (c, 0, 0)),
            pl.BlockSpec((pair, 1, out_lanes), lambda c: (c, 0, 0)),
        ],
        out_shape=[
            jax.ShapeDtypeStruct((num_classes, 1, out_lanes), jnp.float32),
            jax.ShapeDtypeStruct((num_classes, 4, out_lanes), jnp.float32),
            jax.ShapeDtypeStruct((num_classes, 1, out_lanes), jnp.int32),
        ],
        scratch_shapes=[
            pltpu.VMEM((pair, rows, _LANES), jnp.float32),
            pltpu.VMEM((rows, _LANES), jnp.float32),
            pltpu.VMEM((rows, _LANES), jnp.float32),
            pltpu.VMEM((rows, _LANES), jnp.float32),
            pltpu.VMEM((rows, _LANES), jnp.float32),
            pltpu.VMEM((rows, _LANES), jnp.float32),
            pltpu.VMEM((rows, _LANES), jnp.int32),
        ],
        compiler_params=pltpu.CompilerParams(
            dimension_semantics=("parallel",),
        ),
    )(conf_t, anc_t, pr_t)

    sel_scores = sc[:, 0, :max_boxes]
    sel_boxes = jnp.transpose(bx[:, :, :max_boxes], (0, 2, 1))
    class_id = cid[:, 0, :max_boxes]
    return sel_scores, sel_boxes, class_id


def kernel(confidence, anchors_all, pr):
    return _run_nms(confidence, anchors_all, pr,
                    max_boxes=100, score_thr=0.5, iou_thr=0.45)


# speedup vs baseline: 1.7223x; 1.4004x over previous
import jax
import jax.numpy as jnp
from jax import lax
from jax.experimental import pallas as pl
from jax.experimental.pallas import tpu as pltpu

_LANES = 128
_SUB = 8
_CH = 64


def _nms_kernel(conf_ref, anc_ref, pr_ref,
                sc_out, bx_out, cid_out,
                s_ref, y1_ref, x1_ref, y2_ref, x2_ref, area_ref, idx_ref,
                *, max_boxes, score_thr, iou_thr, rows, pair):
    c = pl.program_id(0)
    neg_inf = jnp.float32(-jnp.inf)
    big = jnp.int32(rows * _LANES)
    nch = rows // _CH

    y1_ref[...] = anc_ref[0] + pr_ref[0]
    x1_ref[...] = anc_ref[1] + pr_ref[1]
    y2_ref[...] = anc_ref[2] + pr_ref[2]
    x2_ref[...] = anc_ref[3] + pr_ref[3]
    area_ref[...] = (y2_ref[...] - y1_ref[...]) * (x2_ref[...] - x1_ref[...])
    idx_ref[...] = (lax.broadcasted_iota(jnp.int32, (rows, _LANES), 0) * _LANES
                    + lax.broadcasted_iota(jnp.int32, (rows, _LANES), 1))

    sc_out[...] = jnp.zeros_like(sc_out)
    bx_out[...] = jnp.zeros_like(bx_out)
    cid_out[...] = jnp.zeros_like(cid_out)

    lane1 = lax.broadcasted_iota(jnp.int32, (1, 1, _LANES), 2)
    sub8 = lax.broadcasted_iota(jnp.int32, (_SUB, _LANES), 0)
    lane8 = lax.broadcasted_iota(jnp.int32, (_SUB, _LANES), 1)
    j_iota = lax.broadcasted_iota(jnp.int32, (1, 4, 1), 1)
    piota = lax.broadcasted_iota(jnp.int32, (pair, 1, 1), 0)

    def vec_p(vals):
        acc = vals[-1]
        for p in reversed(range(len(vals) - 1)):
            acc = jnp.where(piota == p, vals[p], acc)
        return acc

    def merge(acc, chunk):
        acc_cm, acc_rh = acc
        ch_cm, ch_rh = chunk
        cm = jnp.maximum(acc_cm, ch_cm)
        rh = jnp.where(acc_cm >= ch_cm, acc_rh, ch_rh)
        return cm, rh

    def chunk_stats(news, idx3_ch):
        cm = jnp.max(news, axis=1, keepdims=True)
        rh = jnp.min(jnp.where(news == cm, idx3_ch, big),
                     axis=1, keepdims=True)
        return cm, rh

    def finalize(cm, rh):
        v = jnp.max(cm, axis=2, keepdims=True)
        idxv = jnp.min(jnp.where(cm == v, rh, big),
                       axis=2, keepdims=True)
        return v, idxv

    acc = None
    for i in range(nch):
        ds = pl.ds(i * _CH, _CH)
        conf_ch = conf_ref[:, ds, :]
        s0 = jnp.where(conf_ch >= score_thr, conf_ch, neg_inf)
        s_ref[:, ds, :] = s0
        st = chunk_stats(s0, idx_ref[ds, :][None])
        acc = st if acc is None else merge(acc, st)
    init = finalize(*acc)

    def body(t, carry):
        v, idxv = carry

        by1s, bx1s, by2s, bx2s = [], [], [], []
        for p in range(pair):
            idx_p = idxv[p, 0, 0]
            r = idx_p // _LANES
            l = idx_p % _LANES
            rbase = pl.multiple_of((r >> 3) << 3, _SUB)
            pick_m = (sub8 == (r & 7)) & (lane8 == l)

            def pick(ref):
                tile = ref[pl.ds(rbase, _SUB), :]
                return jnp.sum(jnp.where(pick_m, tile, 0.0))

            by1s.append(pick(y1_ref))
            bx1s.append(pick(x1_ref))
            by2s.append(pick(y2_ref))
            bx2s.append(pick(x2_ref))

        by1 = vec_p(by1s)
        bx1 = vec_p(bx1s)
        by2 = vec_p(by2s)
        bx2 = vec_p(bx2s)
        keep = v > neg_inf
        area_a = (by2 - by1) * (bx2 - bx1)

        acc = None
        for i in range(nch):
            ds = pl.ds(i * _CH, _CH)
            cy1 = y1_ref[ds, :][None]
            cx1 = x1_ref[ds, :][None]
            cy2 = y2_ref[ds, :][None]
            cx2 = x2_ref[ds, :][None]
            idx3 = idx_ref[ds, :][None]
            s = s_ref[:, ds, :]
            inter = (jnp.maximum(jnp.minimum(by2, cy2) - jnp.maximum(by1, cy1), 0.0)
                     * jnp.maximum(jnp.minimum(bx2, cx2) - jnp.maximum(bx1, cx1), 0.0))
            union = area_a + area_ref[ds, :][None] - inter
            iou = jnp.where(union > 0.0, inter / union, 0.0)
            news = jnp.where((iou > iou_thr) | (idx3 == idxv), neg_inf, s)
            s_ref[:, ds, :] = news
            st = chunk_stats(news, idx3)
            acc = st if acc is None else merge(acc, st)
        nxt = finalize(*acc)

        sel = lane1 == t
        sc_out[...] = jnp.where(sel, jnp.where(keep, v, 0.0), sc_out[...])
        cidv = jnp.where(keep, c * pair + piota + 1, 0)
        cid_out[...] = jnp.where(sel, cidv, cid_out[...])
        coords = jnp.where(j_iota == 0, by1,
                           jnp.where(j_iota == 1, bx1,
                                     jnp.where(j_iota == 2, by2, bx2)))
        coords = jnp.where(keep, coords, 0.0)
        bx_out[...] = jnp.where(sel, coords, bx_out[...])
        return nxt

    lax.fori_loop(0, max_boxes, body, init)


def _run_nms(confidence, anchors_all, pr, max_boxes, score_thr, iou_thr,
             pair=2):
    n, num_classes = confidence.shape
    rows = ((n + _LANES * _CH - 1) // (_LANES * _CH)) * _CH
    n_pad = rows * _LANES
    if num_classes % pair != 0:
        pair = 1

    conf_t = jnp.pad(confidence.T, ((0, 0), (0, n_pad - n)))
    conf_t = conf_t.reshape(num_classes, rows, _LANES)
    anc_t = jnp.pad(anchors_all.T, ((0, 0), (0, n_pad - n))).reshape(4, rows, _LANES)
    pr_t = jnp.pad(pr.T, ((0, 0), (0, n_pad - n))).reshape(4, rows, _LANES)

    out_lanes = ((max_boxes + _LANES - 1) // _LANES) * _LANES

    def kern(*refs):
        return _nms_kernel(*refs, max_boxes=max_boxes, score_thr=score_thr,
                           iou_thr=iou_thr, rows=rows, pair=pair)

    sc, bx, cid = pl.pallas_call(
        kern,
        grid=(num_classes // pair,),
        in_specs=[
            pl.BlockSpec((pair, rows, _LANES), lambda c: (c, 0, 0)),
            pl.BlockSpec((4, rows, _LANES), lambda c: (0, 0, 0)),
            pl.BlockSpec((4, rows, _LANES), lambda c: (0, 0, 0)),
        ],
        out_specs=[
            pl.BlockSpec((pair, 1, out_lanes), lambda c: (c, 0, 0)),
            pl.BlockSpec((pair, 4, out_lanes), lambda c: (c, 0, 0)),
            pl.BlockSpec((pair, 1, out_lanes), lambda c: (c, 0, 0)),
        ],
        out_shape=[
            jax.ShapeDtypeStruct((num_classes, 1, out_lanes), jnp.float32),
            jax.ShapeDtypeStruct((num_classes, 4, out_lanes), jnp.float32),
            jax.ShapeDtypeStruct((num_classes, 1, out_lanes), jnp.int32),
        ],
        scratch_shapes=[
            pltpu.VMEM((pair, rows, _LANES), jnp.float32),
            pltpu.VMEM((rows, _LANES), jnp.float32),
            pltpu.VMEM((rows, _LANES), jnp.float32),
            pltpu.VMEM((rows, _LANES), jnp.float32),
            pltpu.VMEM((rows, _LANES), jnp.float32),
            pltpu.VMEM((rows, _LANES), jnp.float32),
            pltpu.VMEM((rows, _LANES), jnp.int32),
        ],
        compiler_params=pltpu.CompilerParams(
            dimension_semantics=("parallel",),
        ),
    )(conf_t, anc_t, pr_t)

    sel_scores = sc[:, 0, :max_boxes]
    sel_boxes = jnp.transpose(bx[:, :, :max_boxes], (0, 2, 1))
    class_id = cid[:, 0, :max_boxes]
    return sel_scores, sel_boxes, class_id


def kernel(confidence, anchors_all, pr):
    return _run_nms(confidence, anchors_all, pr,
                    max_boxes=100, score_thr=0.5, iou_thr=0.45)


# speedup vs baseline: 1.7262x; 1.0023x over previous
import jax
import jax.numpy as jnp
from jax import lax
from jax.experimental import pallas as pl
from jax.experimental.pallas import tpu as pltpu

_LANES = 128
_SUB = 8
_CH = 32


def _nms_kernel(conf_ref, anc_ref, pr_ref,
                sc_out, bx_out, cid_out,
                s_ref, y1_ref, x1_ref, y2_ref, x2_ref, area_ref, idx_ref,
                *, max_boxes, score_thr, iou_thr, rows, pair):
    c = pl.program_id(0)
    neg_inf = jnp.float32(-jnp.inf)
    big = jnp.int32(rows * _LANES)
    nch = rows // _CH

    y1_ref[...] = anc_ref[0] + pr_ref[0]
    x1_ref[...] = anc_ref[1] + pr_ref[1]
    y2_ref[...] = anc_ref[2] + pr_ref[2]
    x2_ref[...] = anc_ref[3] + pr_ref[3]
    area_ref[...] = (y2_ref[...] - y1_ref[...]) * (x2_ref[...] - x1_ref[...])
    idx_ref[...] = (lax.broadcasted_iota(jnp.int32, (rows, _LANES), 0) * _LANES
                    + lax.broadcasted_iota(jnp.int32, (rows, _LANES), 1))

    sc_out[...] = jnp.zeros_like(sc_out)
    bx_out[...] = jnp.zeros_like(bx_out)
    cid_out[...] = jnp.zeros_like(cid_out)

    lane1 = lax.broadcasted_iota(jnp.int32, (1, 1, _LANES), 2)
    sub8 = lax.broadcasted_iota(jnp.int32, (_SUB, _LANES), 0)
    lane8 = lax.broadcasted_iota(jnp.int32, (_SUB, _LANES), 1)
    j_iota = lax.broadcasted_iota(jnp.int32, (1, 4, 1), 1)
    piota = lax.broadcasted_iota(jnp.int32, (pair, 1, 1), 0)

    def vec_p(vals):
        acc = vals[-1]
        for p in reversed(range(len(vals) - 1)):
            acc = jnp.where(piota == p, vals[p], acc)
        return acc

    def merge(acc, chunk):
        acc_cm, acc_rh = acc
        ch_cm, ch_rh = chunk
        cm = jnp.maximum(acc_cm, ch_cm)
        rh = jnp.where(acc_cm >= ch_cm, acc_rh, ch_rh)
        return cm, rh

    def chunk_stats(news, idx3_ch):
        cm = jnp.max(news, axis=1, keepdims=True)
        rh = jnp.min(jnp.where(news == cm, idx3_ch, big),
                     axis=1, keepdims=True)
        return cm, rh

    def finalize(cm, rh):
        v = jnp.max(cm, axis=2, keepdims=True)
        idxv = jnp.min(jnp.where(cm == v, rh, big),
                       axis=2, keepdims=True)
        return v, idxv

    acc = None
    for i in range(nch):
        ds = pl.ds(i * _CH, _CH)
        conf_ch = conf_ref[:, ds, :]
        s0 = jnp.where(conf_ch >= score_thr, conf_ch, neg_inf)
        s_ref[:, ds, :] = s0
        st = chunk_stats(s0, idx_ref[ds, :][None])
        acc = st if acc is None else merge(acc, st)
    init = finalize(*acc)

    def body(t, carry):
        v, idxv = carry

        by1s, bx1s, by2s, bx2s = [], [], [], []
        for p in range(pair):
            idx_p = idxv[p, 0, 0]
            r = idx_p // _LANES
            l = idx_p % _LANES
            rbase = pl.multiple_of((r >> 3) << 3, _SUB)
            pick_m = (sub8 == (r & 7)) & (lane8 == l)

            def pick(ref):
                tile = ref[pl.ds(rbase, _SUB), :]
                return jnp.sum(jnp.where(pick_m, tile, 0.0))

            by1s.append(pick(y1_ref))
            bx1s.append(pick(x1_ref))
            by2s.append(pick(y2_ref))
            bx2s.append(pick(x2_ref))

        by1 = vec_p(by1s)
        bx1 = vec_p(bx1s)
        by2 = vec_p(by2s)
        bx2 = vec_p(bx2s)
        keep = v > neg_inf
        area_a = (by2 - by1) * (bx2 - bx1)

        acc = None
        for i in range(nch):
            ds = pl.ds(i * _CH, _CH)
            cy1 = y1_ref[ds, :][None]
            cx1 = x1_ref[ds, :][None]
            cy2 = y2_ref[ds, :][None]
            cx2 = x2_ref[ds, :][None]
            idx3 = idx_ref[ds, :][None]
            s = s_ref[:, ds, :]
            inter = (jnp.maximum(jnp.minimum(by2, cy2) - jnp.maximum(by1, cy1), 0.0)
                     * jnp.maximum(jnp.minimum(bx2, cx2) - jnp.maximum(bx1, cx1), 0.0))
            union = area_a + area_ref[ds, :][None] - inter
            iou = jnp.where(union > 0.0, inter / union, 0.0)
            news = jnp.where((iou > iou_thr) | (idx3 == idxv), neg_inf, s)
            s_ref[:, ds, :] = news
            st = chunk_stats(news, idx3)
            acc = st if acc is None else merge(acc, st)
        nxt = finalize(*acc)

        sel = lane1 == t
        sc_out[...] = jnp.where(sel, jnp.where(keep, v, 0.0), sc_out[...])
        cidv = jnp.where(keep, c * pair + piota + 1, 0)
        cid_out[...] = jnp.where(sel, cidv, cid_out[...])
        coords = jnp.where(j_iota == 0, by1,
                           jnp.where(j_iota == 1, bx1,
                                     jnp.where(j_iota == 2, by2, bx2)))
        coords = jnp.where(keep, coords, 0.0)
        bx_out[...] = jnp.where(sel, coords, bx_out[...])
        return nxt

    lax.fori_loop(0, max_boxes, body, init)


def _run_nms(confidence, anchors_all, pr, max_boxes, score_thr, iou_thr,
             pair=2):
    n, num_classes = confidence.shape
    rows = ((n + _LANES * _CH - 1) // (_LANES * _CH)) * _CH
    n_pad = rows * _LANES
    if num_classes % pair != 0:
        pair = 1

    conf_t = jnp.pad(confidence.T, ((0, 0), (0, n_pad - n)))
    conf_t = conf_t.reshape(num_classes, rows, _LANES)
    anc_t = jnp.pad(anchors_all.T, ((0, 0), (0, n_pad - n))).reshape(4, rows, _LANES)
    pr_t = jnp.pad(pr.T, ((0, 0), (0, n_pad - n))).reshape(4, rows, _LANES)

    out_lanes = ((max_boxes + _LANES - 1) // _LANES) * _LANES

    def kern(*refs):
        return _nms_kernel(*refs, max_boxes=max_boxes, score_thr=score_thr,
                           iou_thr=iou_thr, rows=rows, pair=pair)

    sc, bx, cid = pl.pallas_call(
        kern,
        grid=(num_classes // pair,),
        in_specs=[
            pl.BlockSpec((pair, rows, _LANES), lambda c: (c, 0, 0)),
            pl.BlockSpec((4, rows, _LANES), lambda c: (0, 0, 0)),
            pl.BlockSpec((4, rows, _LANES), lambda c: (0, 0, 0)),
        ],
        out_specs=[
            pl.BlockSpec((pair, 1, out_lanes), lambda c: (c, 0, 0)),
            pl.BlockSpec((pair, 4, out_lanes), lambda c: (c, 0, 0)),
            pl.BlockSpec((pair, 1, out_lanes), lambda c: (c, 0, 0)),
        ],
        out_shape=[
            jax.ShapeDtypeStruct((num_classes, 1, out_lanes), jnp.float32),
            jax.ShapeDtypeStruct((num_classes, 4, out_lanes), jnp.float32),
            jax.ShapeDtypeStruct((num_classes, 1, out_lanes), jnp.int32),
        ],
        scratch_shapes=[
            pltpu.VMEM((pair, rows, _LANES), jnp.float32),
            pltpu.VMEM((rows, _LANES), jnp.float32),
            pltpu.VMEM((rows, _LANES), jnp.float32),
            pltpu.VMEM((rows, _LANES), jnp.float32),
            pltpu.VMEM((rows, _LANES), jnp.float32),
            pltpu.VMEM((rows, _LANES), jnp.float32),
            pltpu.VMEM((rows, _LANES), jnp.int32),
        ],
        compiler_params=pltpu.CompilerParams(
            dimension_semantics=("parallel",),
        ),
    )(conf_t, anc_t, pr_t)

    sel_scores = sc[:, 0, :max_boxes]
    sel_boxes = jnp.transpose(bx[:, :, :max_boxes], (0, 2, 1))
    class_id = cid[:, 0, :max_boxes]
    return sel_scores, sel_boxes, class_id


def kernel(confidence, anchors_all, pr):
    return _run_nms(confidence, anchors_all, pr,
                    max_boxes=100, score_thr=0.5, iou_thr=0.45)


# speedup vs baseline: 1.8978x; 1.0994x over previous
import jax
import jax.numpy as jnp
from jax import lax
from jax.experimental import pallas as pl
from jax.experimental.pallas import tpu as pltpu

_LANES = 128
_SUB = 8
_CH = 32


def _nms_kernel(conf_ref, anc_ref, pr_ref,
                sc_out, bx_out, cid_out,
                s_ref, y1_ref, x1_ref, y2_ref, x2_ref, area_ref, idx_ref,
                *, max_boxes, score_thr, iou_thr, rows, pair):
    c = pl.program_id(0)
    neg_inf = jnp.float32(-jnp.inf)
    big = jnp.int32(rows * _LANES)
    nch = rows // _CH

    y1_ref[...] = anc_ref[0] + pr_ref[0]
    x1_ref[...] = anc_ref[1] + pr_ref[1]
    y2_ref[...] = anc_ref[2] + pr_ref[2]
    x2_ref[...] = anc_ref[3] + pr_ref[3]
    area_ref[...] = (y2_ref[...] - y1_ref[...]) * (x2_ref[...] - x1_ref[...])
    idx_ref[...] = (lax.broadcasted_iota(jnp.int32, (rows, _LANES), 0) * _LANES
                    + lax.broadcasted_iota(jnp.int32, (rows, _LANES), 1))

    sc_out[...] = jnp.zeros_like(sc_out)
    bx_out[...] = jnp.zeros_like(bx_out)
    cid_out[...] = jnp.zeros_like(cid_out)

    lane1 = lax.broadcasted_iota(jnp.int32, (1, 1, _LANES), 2)
    sub8 = lax.broadcasted_iota(jnp.int32, (_SUB, _LANES), 0)
    lane8 = lax.broadcasted_iota(jnp.int32, (_SUB, _LANES), 1)
    j_iota = lax.broadcasted_iota(jnp.int32, (1, 4, 1), 1)
    piota = lax.broadcasted_iota(jnp.int32, (pair, 1, 1), 0)

    def vec_p(vals):
        acc = vals[-1]
        for p in reversed(range(len(vals) - 1)):
            acc = jnp.where(piota == p, vals[p], acc)
        return acc

    def merge(acc, chunk):
        acc_cm, acc_rh = acc
        ch_cm, ch_rh = chunk
        cm = jnp.maximum(acc_cm, ch_cm)
        rh = jnp.where(acc_cm >= ch_cm, acc_rh, ch_rh)
        return cm, rh

    def chunk_stats(news, idx3_ch):
        cm = jnp.max(news, axis=1, keepdims=True)
        rh = jnp.min(jnp.where(news == cm, idx3_ch, big),
                     axis=1, keepdims=True)
        return cm, rh

    def finalize(cm, rh):
        v = jnp.max(cm, axis=2, keepdims=True)
        idxv = jnp.min(jnp.where(cm == v, rh, big),
                       axis=2, keepdims=True)
        return v, idxv

    acc = None
    for i in range(nch):
        ds = pl.ds(i * _CH, _CH)
        conf_ch = conf_ref[:, ds, :]
        s0 = jnp.where(conf_ch >= score_thr, conf_ch, neg_inf)
        s_ref[:, ds, :] = s0
        st = chunk_stats(s0, idx_ref[ds, :][None])
        acc = st if acc is None else merge(acc, st)
    init = finalize(*acc)

    def body(t, carry):
        v, idxv = carry

        by1s, bx1s, by2s, bx2s = [], [], [], []
        for p in range(pair):
            idx_p = idxv[p, 0, 0]
            r = idx_p // _LANES
            l = idx_p % _LANES
            rbase = pl.multiple_of((r >> 3) << 3, _SUB)
            pick_m = (sub8 == (r & 7)) & (lane8 == l)

            def pick(ref):
                tile = ref[pl.ds(rbase, _SUB), :]
                return jnp.sum(jnp.where(pick_m, tile, 0.0))

            by1s.append(pick(y1_ref))
            bx1s.append(pick(x1_ref))
            by2s.append(pick(y2_ref))
            bx2s.append(pick(x2_ref))

        by1 = vec_p(by1s)
        bx1 = vec_p(bx1s)
        by2 = vec_p(by2s)
        bx2 = vec_p(bx2s)
        keep = v > neg_inf
        area_a = (by2 - by1) * (bx2 - bx1)

        acc = None
        for i in range(nch):
            ds = pl.ds(i * _CH, _CH)
            cy1 = y1_ref[ds, :][None]
            cx1 = x1_ref[ds, :][None]
            cy2 = y2_ref[ds, :][None]
            cx2 = x2_ref[ds, :][None]
            idx3 = idx_ref[ds, :][None]
            s = s_ref[:, ds, :]
            inter = (jnp.maximum(jnp.minimum(by2, cy2) - jnp.maximum(by1, cy1), 0.0)
                     * jnp.maximum(jnp.minimum(bx2, cx2) - jnp.maximum(bx1, cx1), 0.0))
            union = area_a + area_ref[ds, :][None] - inter
            iou = jnp.where(union > 0.0, inter / union, 0.0)
            news = jnp.where((iou > iou_thr) | (idx3 == idxv), neg_inf, s)
            s_ref[:, ds, :] = news
            st = chunk_stats(news, idx3)
            acc = st if acc is None else merge(acc, st)
        nxt = finalize(*acc)

        sel = lane1 == t
        sc_out[...] = jnp.where(sel, jnp.where(keep, v, 0.0), sc_out[...])
        cidv = jnp.where(keep, c * pair + piota + 1, 0)
        cid_out[...] = jnp.where(sel, cidv, cid_out[...])
        coords = jnp.where(j_iota == 0, by1,
                           jnp.where(j_iota == 1, bx1,
                                     jnp.where(j_iota == 2, by2, bx2)))
        coords = jnp.where(keep, coords, 0.0)
        bx_out[...] = jnp.where(sel, coords, bx_out[...])
        return nxt

    lax.fori_loop(0, max_boxes, body, init)


def _run_nms(confidence, anchors_all, pr, max_boxes, score_thr, iou_thr,
             pair=4):
    n, num_classes = confidence.shape
    rows = ((n + _LANES * _CH - 1) // (_LANES * _CH)) * _CH
    n_pad = rows * _LANES
    if num_classes % pair != 0:
        pair = 1

    conf_t = jnp.pad(confidence.T, ((0, 0), (0, n_pad - n)))
    conf_t = conf_t.reshape(num_classes, rows, _LANES)
    anc_t = jnp.pad(anchors_all.T, ((0, 0), (0, n_pad - n))).reshape(4, rows, _LANES)
    pr_t = jnp.pad(pr.T, ((0, 0), (0, n_pad - n))).reshape(4, rows, _LANES)

    out_lanes = ((max_boxes + _LANES - 1) // _LANES) * _LANES

    def kern(*refs):
        return _nms_kernel(*refs, max_boxes=max_boxes, score_thr=score_thr,
                           iou_thr=iou_thr, rows=rows, pair=pair)

    sc, bx, cid = pl.pallas_call(
        kern,
        grid=(num_classes // pair,),
        in_specs=[
            pl.BlockSpec((pair, rows, _LANES), lambda c: (c, 0, 0)),
            pl.BlockSpec((4, rows, _LANES), lambda c: (0, 0, 0)),
            pl.BlockSpec((4, rows, _LANES), lambda c: (0, 0, 0)),
        ],
        out_specs=[
            pl.BlockSpec((pair, 1, out_lanes), lambda c: (c, 0, 0)),
            pl.BlockSpec((pair, 4, out_lanes), lambda c: (c, 0, 0)),
            pl.BlockSpec((pair, 1, out_lanes), lambda c: (c, 0, 0)),
        ],
        out_shape=[
            jax.ShapeDtypeStruct((num_classes, 1, out_lanes), jnp.float32),
            jax.ShapeDtypeStruct((num_classes, 4, out_lanes), jnp.float32),
            jax.ShapeDtypeStruct((num_classes, 1, out_lanes), jnp.int32),
        ],
        scratch_shapes=[
            pltpu.VMEM((pair, rows, _LANES), jnp.float32),
            pltpu.VMEM((rows, _LANES), jnp.float32),
            pltpu.VMEM((rows, _LANES), jnp.float32),
            pltpu.VMEM((rows, _LANES), jnp.float32),
            pltpu.VMEM((rows, _LANES), jnp.float32),
            pltpu.VMEM((rows, _LANES), jnp.float32),
            pltpu.VMEM((rows, _LANES), jnp.int32),
        ],
        compiler_params=pltpu.CompilerParams(
            dimension_semantics=("parallel",),
        ),
    )(conf_t, anc_t, pr_t)

    sel_scores = sc[:, 0, :max_boxes]
    sel_boxes = jnp.transpose(bx[:, :, :max_boxes], (0, 2, 1))
    class_id = cid[:, 0, :max_boxes]
    return sel_scores, sel_boxes, class_id


def kernel(confidence, anchors_all, pr):
    return _run_nms(confidence, anchors_all, pr,
                    max_boxes=100, score_thr=0.5, iou_thr=0.45)
